# baseline (speedup 1.0000x reference)
import dataclasses

import jax
import jax.numpy as jnp
from jax import lax
from jax.experimental import pallas as pl
from jax.experimental.pallas import tpu as pltpu
from jax.experimental.pallas import tpu_sc as plsc

DIM = 128
NV = DIM // 16
NC = 2
NS = 16
NW = NC * NS
LOG_CHE = 7
CHE = 1 << LOG_CHE
NSUB = CHE // 128
D = 6


def _bag_mean_sc(table, ids_ext, off_ext, b):
    bpw = b // NW
    mesh = plsc.VectorSubcoreMesh(core_axis_name="c", subcore_axis_name="s",
                                  num_cores=NC, num_subcores=NS)

    def body(table_hbm, ids_hbm, off_hbm, out_hbm, *scr):
        idxs = scr[0:D]
        rows = scr[D:2 * D]
        outbuf_v = scr[2 * D]
        off_v = scr[2 * D + 1]
        sm = scr[2 * D + 2]
        sgs = scr[2 * D + 3:2 * D + 3 + D]
        sis = scr[2 * D + 3 + D:2 * D + 3 + 2 * D]

        w = lax.axis_index("s") * NC + lax.axis_index("c")
        jbeg = pl.multiple_of(w * bpw, 8)
        pltpu.sync_copy(off_hbm.at[pl.ds(jbeg, bpw + 16)], off_v)

        def ext(j, carry):
            g = plsc.load_gather(off_v, [jnp.full((16,), j, jnp.int32)])
            sm[j] = lax.reduce_max(g, axes=(0,))
            return carry

        lax.fori_loop(0, bpw + 1, ext, jnp.int32(0))
        ws = sm[0]
        we = sm[bpw]
        c0w = lax.shift_right_logical(ws, LOG_CHE)
        nch = lax.select(
            we > ws,
            lax.shift_right_logical(we - 1, LOG_CHE) - c0w + 1,
            jnp.int32(0))

        def cbase_of(ci):
            return pl.multiple_of(lax.shift_left(c0w + ci, LOG_CHE), 8)

        def ids_issue(ci, which):
            pltpu.async_copy(ids_hbm.at[pl.ds(cbase_of(ci), CHE)],
                             idxs[which], sis[which])

        def ids_wait(ci, which):
            pltpu.make_async_copy(ids_hbm.at[pl.ds(cbase_of(ci), CHE)],
                                  idxs[which], sis[which]).wait()

        def gather_issue(which):
            for k in range(NSUB):
                pltpu.async_copy(
                    table_hbm.at[idxs[which].at[pl.ds(k * 128, 128)]],
                    rows[which].at[pl.ds(k * 128, 128)], sgs[which])

        def gather_wait(which):
            for k in range(NSUB):
                pltpu.make_async_copy(
                    table_hbm.at[idxs[which].at[pl.ds(k * 128, 128)]],
                    rows[which].at[pl.ds(k * 128, 128)], sgs[which]).wait()

        def switch(m, mk, carry):
            def rec(lo, hi):
                if hi - lo == 1:
                    return mk(lo)
                mid = (lo + hi) // 2
                return lambda c: lax.cond(m < mid, rec(lo, mid),
                                          rec(mid, hi), c)
            return rec(0, D)(carry)

        for d in range(D):
            @pl.when(nch > d)
            def _(d=d):
                ids_issue(jnp.int32(d), d)
        for d in range(D - 1):
            @pl.when(nch > d)
            def _(d=d):
                ids_wait(jnp.int32(d), d)
                gather_issue(d)

        def event(ci):
            def mk(i):
                def go(carry):
                    gather_wait(i)

                    @pl.when(ci + (D - 1) < nch)
                    def _():
                        ids_wait(ci + (D - 1), (i + D - 1) % D)
                        gather_issue((i + D - 1) % D)

                    @pl.when(ci + D < nch)
                    def _():
                        ids_issue(ci + D, i)
                    return carry
                return go

            switch(lax.rem(ci, D), mk, jnp.int32(0))

        acc0 = (jnp.zeros((16,), jnp.float32),) * NV

        def accum(m, lo, hi, acc):
            def mk(which):
                def go(acc):
                    def rbody(r, a):
                        return tuple(
                            a[v] + rows[which][r, pl.ds(v * 16, 16)]
                            for v in range(NV))
                    return lax.fori_loop(lo, hi, rbody, acc)
                return go
            return switch(m, mk, acc)

        def bag_body(j, loaded):
            s = sm[j]
            e = sm[j + 1]
            c0 = lax.shift_right_logical(s, LOG_CHE)
            c1 = lax.shift_right_logical(lax.max(e, s + 1) - 1, LOG_CHE)
            nspan = lax.select(e > s, c1 - c0 + 1, jnp.int32(0))

            def chunk_body(cc, carry):
                loaded, acc = carry
                ci = cc - c0w

                def load(c):
                    event(ci)
                    return cc

                loaded = lax.cond(cc != loaded, load, lambda c: c, loaded)
                cbase = pl.multiple_of(lax.shift_left(cc, LOG_CHE), 8)
                lo = lax.max(s, cbase) - cbase
                hi = lax.min(e, cbase + CHE) - cbase
                acc = accum(lax.rem(ci, D), lo, hi, acc)
                return (loaded, acc)

            loaded, acc = lax.fori_loop(c0, c0 + nspan, chunk_body,
                                        (loaded, acc0))
            cntf = lax.convert_element_type(lax.max(e - s, 1), jnp.float32)
            scale = jnp.full((16,), 1.0, jnp.float32) / jnp.full(
                (16,), cntf, jnp.float32)
            for v in range(NV):
                outbuf_v[j, pl.ds(v * 16, 16)] = acc[v] * scale
            return loaded

        lax.fori_loop(0, bpw, bag_body, jnp.int32(-1))
        pltpu.sync_copy(outbuf_v, out_hbm.at[pl.ds(jbeg, bpw)])

    cp = pltpu.CompilerParams()
    if "needs_layout_passes" in pltpu.CompilerParams.__dataclass_fields__:
        cp = dataclasses.replace(cp, needs_layout_passes=False)
    kern = pl.kernel(
        body,
        out_type=jax.ShapeDtypeStruct((b, DIM), jnp.float32),
        mesh=mesh,
        compiler_params=cp,
        scratch_types=(
            [pltpu.VMEM((CHE,), jnp.int32) for _ in range(D)]
            + [pltpu.VMEM((CHE, DIM), jnp.float32) for _ in range(D)]
            + [pltpu.VMEM((bpw, DIM), jnp.float32),
               pltpu.VMEM((bpw + 16,), jnp.int32),
               pltpu.SMEM((bpw + 16,), jnp.int32)]
            + [pltpu.SemaphoreType.DMA for _ in range(2 * D)]
        ),
    )
    return kern(table, ids_ext, off_ext)


@jax.jit
def kernel(ids, offsets, table):
    n = ids.shape[0]
    b = offsets.shape[0]
    ids_ext = jnp.concatenate(
        [ids.astype(jnp.int32), jnp.zeros((CHE,), jnp.int32)])
    pad = (-ids_ext.shape[0]) % CHE
    if pad:
        ids_ext = jnp.concatenate([ids_ext, jnp.zeros((pad,), jnp.int32)])
    off_ext = jnp.concatenate(
        [offsets.astype(jnp.int32), jnp.full((16,), n, jnp.int32)])
    return _bag_mean_sc(table, ids_ext, off_ext, b)


# speedup vs baseline: 1.0080x; 1.0080x over previous
import dataclasses

import jax
import jax.numpy as jnp
from jax import lax
from jax.experimental import pallas as pl
from jax.experimental.pallas import tpu as pltpu
from jax.experimental.pallas import tpu_sc as plsc

DIM = 128
NV = DIM // 16
NC = 2
NS = 16
NW = NC * NS
LOG_CHE = 8
CHE = 1 << LOG_CHE
NSUB = CHE // 128
D = 3
SPILL = 64


def _bag_mean_sc(table, ids_ext, off_ext, n, b):
    mesh = plsc.VectorSubcoreMesh(core_axis_name="c", subcore_axis_name="s",
                                  num_cores=NC, num_subcores=NS)
    epw = (n + NW - 1) // NW
    nboff = off_ext.shape[0]

    def body(table_hbm, ids_hbm, off_hbm, out_hbm,
             idx0, idx1, idx2, rows0, rows1, rows2, outbuf_v, off_v,
             sg0, sg1, sg2, si0, si1, si2, so, sr):
        w = lax.axis_index("s") * NC + lax.axis_index("c")
        pltpu.sync_copy(off_hbm, off_v)

        def ext(j):
            g = plsc.load_gather(off_v, [jnp.full((16,), j, jnp.int32)])
            return lax.reduce_max(g, axes=(0,))

        def lower_bound(target):
            def step(_, c):
                lo, hi = c
                mid = lax.shift_right_logical(lo + hi, 1)
                v = ext(mid)
                return (lax.select(v < target, mid + 1, lo),
                        lax.select(v < target, hi, mid))
            lo, _ = lax.fori_loop(0, 13, step, (jnp.int32(0), jnp.int32(b)))
            return lo

        jw = pl.multiple_of(lower_bound(w * epw) & (-8), 8)
        jnext = pl.multiple_of(
            lax.select(w == NW - 1, jnp.int32(b),
                       lower_bound((w + 1) * epw) & (-8)), 8)
        ws = ext(jw)
        we = ext(jnext)
        c0w = lax.shift_right_logical(ws, LOG_CHE)
        nch = lax.select(
            we > ws,
            lax.shift_right_logical(we - 1, LOG_CHE) - c0w + 1,
            jnp.int32(0))

        idxs = (idx0, idx1, idx2)
        rows = (rows0, rows1, rows2)
        sgs = (sg0, sg1, sg2)
        sis = (si0, si1, si2)

        def cbase_of(ci):
            return pl.multiple_of(lax.shift_left(c0w + ci, LOG_CHE), 8)

        def ids_issue(ci, which):
            pltpu.async_copy(ids_hbm.at[pl.ds(cbase_of(ci), CHE)],
                             idxs[which], sis[which])

        def ids_wait(ci, which):
            pltpu.make_async_copy(ids_hbm.at[pl.ds(cbase_of(ci), CHE)],
                                  idxs[which], sis[which]).wait()

        def gather_issue(which):
            for k in range(NSUB):
                pltpu.async_copy(
                    table_hbm.at[idxs[which].at[pl.ds(k * 128, 128)]],
                    rows[which].at[pl.ds(k * 128, 128)], sgs[which])

        def gather_wait(which):
            for k in range(NSUB):
                pltpu.make_async_copy(
                    table_hbm.at[idxs[which].at[pl.ds(k * 128, 128)]],
                    rows[which].at[pl.ds(k * 128, 128)], sgs[which]).wait()

        def switch(m, mk, carry):
            return lax.cond(
                m == 0, mk(0),
                lambda c: lax.cond(m == 1, mk(1), mk(2), c),
                carry)

        for d in range(D):
            @pl.when(nch > d)
            def _(d=d):
                ids_issue(jnp.int32(d), d)
        for d in range(D - 1):
            @pl.when(nch > d)
            def _(d=d):
                ids_wait(jnp.int32(d), d)
                gather_issue(d)

        def event(ci):
            def mk(i):
                def go(carry):
                    gather_wait(i)

                    @pl.when(ci + 2 < nch)
                    def _():
                        ids_wait(ci + 2, (i + 2) % D)
                        gather_issue((i + 2) % D)

                    @pl.when(ci + 3 < nch)
                    def _():
                        ids_issue(ci + 3, i)
                    return carry
                return go

            switch(lax.rem(ci, D), mk, jnp.int32(0))

        acc0 = (jnp.zeros((16,), jnp.float32),) * NV

        def accum(m, lo, hi, acc):
            def mk(which):
                def go(acc):
                    def rbody(r, a):
                        return tuple(
                            a[v] + rows[which][r, pl.ds(v * 16, 16)]
                            for v in range(NV))
                    return lax.fori_loop(lo, hi, rbody, acc)
                return go
            return switch(m, mk, acc)

        def spill_wait():
            pltpu.make_async_copy(
                outbuf_v.at[pl.ds(0, SPILL)],
                out_hbm.at[pl.ds(0, SPILL)], so).wait()

        def bag_body(j, carry):
            loaded, s = carry
            e = ext(j + 1)
            c0 = lax.shift_right_logical(s, LOG_CHE)
            c1 = lax.shift_right_logical(lax.max(e, s + 1) - 1, LOG_CHE)
            nspan = lax.select(e > s, c1 - c0 + 1, jnp.int32(0))

            def chunk_body(cc, carry):
                loaded, acc = carry
                ci = cc - c0w

                def load(c):
                    event(ci)
                    return cc

                loaded = lax.cond(cc != loaded, load, lambda c: c, loaded)
                cbase = pl.multiple_of(lax.shift_left(cc, LOG_CHE), 8)
                lo = lax.max(s, cbase) - cbase
                hi = lax.min(e, cbase + CHE) - cbase
                acc = accum(lax.rem(ci, D), lo, hi, acc)
                return (loaded, acc)

            loaded, acc = lax.fori_loop(c0, c0 + nspan, chunk_body,
                                        (loaded, acc0))
            cntf = lax.convert_element_type(lax.max(e - s, 1), jnp.float32)
            scale = jnp.full((16,), 1.0, jnp.float32) / jnp.full(
                (16,), cntf, jnp.float32)
            nb = j - jw
            slot = (nb & (SPILL - 1)) + (
                lax.shift_right_logical(nb, 6) & 1) * SPILL
            for v in range(NV):
                outbuf_v[slot, pl.ds(v * 16, 16)] = acc[v] * scale

            @pl.when((nb & (SPILL - 1)) == SPILL - 1)
            def _():
                halfbase = pl.multiple_of(
                    (lax.shift_right_logical(nb, 6) & 1) * SPILL, 8)

                @pl.when(nb >= 2 * SPILL - 1)
                def _():
                    spill_wait()
                pltpu.async_copy(
                    outbuf_v.at[pl.ds(halfbase, SPILL)],
                    out_hbm.at[pl.ds(
                        pl.multiple_of(j - (SPILL - 1), 8), SPILL)], so)
            return (loaded, e)

        _, _ = lax.fori_loop(jw, jnext, bag_body, (jnp.int32(-1), ws))

        nbags = jnext - jw
        tail = nbags & (SPILL - 1)
        ngrp = lax.shift_right_logical(tail, 3)
        halfbase = pl.multiple_of(
            (lax.shift_right_logical(nbags, 6) & 1) * SPILL, 8)

        def tail_issue(r, c):
            r8 = lax.shift_left(r, 3)
            pltpu.async_copy(
                outbuf_v.at[pl.ds(pl.multiple_of(halfbase + r8, 8), 8)],
                out_hbm.at[pl.ds(
                    pl.multiple_of(jnext - tail + r8, 8), 8)], sr)
            return c

        lax.fori_loop(0, ngrp, tail_issue, jnp.int32(0))

        @pl.when(nbags >= SPILL)
        def _():
            spill_wait()

        def tail_drain(r, c):
            pltpu.make_async_copy(outbuf_v.at[pl.ds(0, 8)],
                                  out_hbm.at[pl.ds(0, 8)], sr).wait()
            return c

        lax.fori_loop(0, ngrp, tail_drain, jnp.int32(0))

    cp = pltpu.CompilerParams()
    if "needs_layout_passes" in pltpu.CompilerParams.__dataclass_fields__:
        cp = dataclasses.replace(cp, needs_layout_passes=False)
    kern = pl.kernel(
        body,
        out_type=jax.ShapeDtypeStruct((b, DIM), jnp.float32),
        mesh=mesh,
        compiler_params=cp,
        scratch_types=[
            pltpu.VMEM((CHE,), jnp.int32),
            pltpu.VMEM((CHE,), jnp.int32),
            pltpu.VMEM((CHE,), jnp.int32),
            pltpu.VMEM((CHE, DIM), jnp.float32),
            pltpu.VMEM((CHE, DIM), jnp.float32),
            pltpu.VMEM((CHE, DIM), jnp.float32),
            pltpu.VMEM((2 * SPILL, DIM), jnp.float32),
            pltpu.VMEM((nboff,), jnp.int32),
            pltpu.SemaphoreType.DMA,
            pltpu.SemaphoreType.DMA,
            pltpu.SemaphoreType.DMA,
            pltpu.SemaphoreType.DMA,
            pltpu.SemaphoreType.DMA,
            pltpu.SemaphoreType.DMA,
            pltpu.SemaphoreType.DMA,
            pltpu.SemaphoreType.DMA,
        ],
    )
    return kern(table, ids_ext, off_ext)


@jax.jit
def kernel(ids, offsets, table):
    n = ids.shape[0]
    b = offsets.shape[0]
    ids_ext = jnp.concatenate(
        [ids.astype(jnp.int32), jnp.zeros((CHE,), jnp.int32)])
    pad = (-ids_ext.shape[0]) % CHE
    if pad:
        ids_ext = jnp.concatenate([ids_ext, jnp.zeros((pad,), jnp.int32)])
    off_ext = jnp.concatenate(
        [offsets.astype(jnp.int32), jnp.full((16,), n, jnp.int32)])
    return _bag_mean_sc(table, ids_ext, off_ext, n, b)
